# Initial kernel scaffold; baseline (speedup 1.0000x reference)
import jax
import jax.numpy as jnp
from jax.experimental import pallas as pl


def kernel(features, src_t2e, dst_t2e, src_e2t, dst_e2t, entity_embed, W_t2e_0, b_t2e_0, W_e2t_0, b_e2t_0, W_t2e_1, b_t2e_1, W_e2t_1, b_e2t_1, W_out, b_out):
    raise NotImplementedError("write your pallas kernel here")


# speedup vs baseline: 28.2651x; 28.2651x over previous
import jax
import jax.numpy as jnp
from jax import lax
from jax.experimental import pallas as pl
from jax.experimental.pallas import tpu as pltpu
from jax.experimental.pallas import tpu_sc as plsc

N_NODES = 50000
N_PAD = 51200
N_EDGES = 1600000
HID = 16
N_CORES = 2
N_SUBCORES = 16
N_WORKERS = N_CORES * N_SUBCORES
E_PER_W = N_EDGES // N_WORKERS
CHUNK = 2000
N_CHUNKS = E_PER_W // CHUNK
ROWS_PER_SUB = N_PAD // N_SUBCORES
IO_CHUNKS = ((0, 1600), (1600, 1600))


def _make_seg_sum():
    mesh = plsc.VectorSubcoreMesh(
        core_axis_name="c", subcore_axis_name="s",
        num_cores=N_CORES, num_subcores=N_SUBCORES)

    def body(table, src, dst, sums_out, cnt_out,
             src_v, dst_v, rows_v, ones_v, acc_sh, cnt_sh, sem):
        cid = lax.axis_index("c")
        sid = lax.axis_index("s")
        wid = cid * N_SUBCORES + sid

        zeros16 = jnp.zeros((16,), jnp.float32)
        ones16 = jnp.ones((16,), jnp.float32)

        def zrow(i, _):
            rows_v[i, :] = zeros16
            return 0
        lax.fori_loop(0, CHUNK, zrow, 0)

        def zrow1(i, _):
            ones_v[pl.ds(i * 16, 16)] = zeros16
            return 0
        lax.fori_loop(0, CHUNK // 16, zrow1, 0)

        rbase = sid * ROWS_PER_SUB
        for off, n in IO_CHUNKS:
            pltpu.sync_copy(rows_v.at[pl.ds(0, n)],
                            acc_sh.at[pl.ds(rbase + off, n)])
            pltpu.sync_copy(ones_v.at[pl.ds(0, n)],
                            cnt_sh.at[pl.ds(rbase + off, n)])

        def orow(i, _):
            ones_v[pl.ds(i * 16, 16)] = ones16
            return 0
        lax.fori_loop(0, CHUNK // 16, orow, 0)
        plsc.subcore_barrier()

        def chunk_body(i, _):
            base = wid * E_PER_W + i * CHUNK
            pltpu.sync_copy(src.at[pl.ds(base, CHUNK)], src_v)
            pltpu.sync_copy(dst.at[pl.ds(base, CHUNK)], dst_v)
            pltpu.async_copy(table.at[src_v], rows_v, sem).wait()
            pltpu.sync_copy(rows_v, acc_sh.at[dst_v], add=True)
            pltpu.sync_copy(ones_v, cnt_sh.at[dst_v], add=True)
            return 0
        lax.fori_loop(0, N_CHUNKS, chunk_body, 0)
        plsc.subcore_barrier()

        for off, n in IO_CHUNKS:
            pltpu.sync_copy(acc_sh.at[pl.ds(rbase + off, n)],
                            rows_v.at[pl.ds(0, n)])
            pltpu.sync_copy(rows_v.at[pl.ds(0, n)],
                            sums_out.at[cid, pl.ds(rbase + off, n)])
            pltpu.sync_copy(cnt_sh.at[pl.ds(rbase + off, n)],
                            ones_v.at[pl.ds(0, n)])
            pltpu.sync_copy(ones_v.at[pl.ds(0, n)],
                            cnt_out.at[cid, pl.ds(rbase + off, n)])

    return pl.kernel(
        body,
        out_type=[
            jax.ShapeDtypeStruct((N_CORES, N_PAD, HID), jnp.float32),
            jax.ShapeDtypeStruct((N_CORES, N_PAD), jnp.float32),
        ],
        mesh=mesh,
        compiler_params=pltpu.CompilerParams(use_tc_tiling_on_sc=False),
        scratch_types=[
            pltpu.VMEM((CHUNK,), jnp.int32),
            pltpu.VMEM((CHUNK,), jnp.int32),
            pltpu.VMEM((CHUNK, HID), jnp.float32),
            pltpu.VMEM((CHUNK,), jnp.float32),
            pltpu.VMEM_SHARED((N_PAD, HID), jnp.float32),
            pltpu.VMEM_SHARED((N_PAD,), jnp.float32),
            pltpu.SemaphoreType.DMA,
        ],
    )


_seg_sum = _make_seg_sum()


def _mm1_kernel(x_ref, w_ref, b_ref, o_ref):
    o_ref[...] = (
        jnp.dot(x_ref[...], w_ref[...], preferred_element_type=jnp.float32)
        + b_ref[...]
    )


_MM1_BLOCK = 2000


def _mm1(features, w, b):
    grid = N_NODES // _MM1_BLOCK
    return pl.pallas_call(
        _mm1_kernel,
        grid=(grid,),
        in_specs=[
            pl.BlockSpec((_MM1_BLOCK, 128), lambda i: (i, 0)),
            pl.BlockSpec((128, HID), lambda i: (0, 0)),
            pl.BlockSpec((1, HID), lambda i: (0, 0)),
        ],
        out_specs=pl.BlockSpec((_MM1_BLOCK, HID), lambda i: (i, 0)),
        out_shape=jax.ShapeDtypeStruct((N_NODES, HID), jnp.float32),
    )(features, w, b)


_CBLK = 3200
_CGRID = N_PAD // _CBLK


def _comb1_kernel(s_ref, c_ref, w_ref, b_ref, o_ref):
    s = s_ref[0] + s_ref[1]
    c = jnp.maximum(c_ref[0] + c_ref[1], 1.0)
    h = s / c[:, None]
    h = jnp.where(h >= 0, h, 0.01 * h)
    o_ref[...] = (
        jnp.dot(h, w_ref[...], preferred_element_type=jnp.float32)
        + b_ref[...]
    )


def _comb1(sums, cnt, w, b):
    return pl.pallas_call(
        _comb1_kernel,
        grid=(_CGRID,),
        in_specs=[
            pl.BlockSpec((N_CORES, _CBLK, HID), lambda i: (0, i, 0)),
            pl.BlockSpec((N_CORES, _CBLK), lambda i: (0, i)),
            pl.BlockSpec((HID, HID), lambda i: (0, 0)),
            pl.BlockSpec((1, HID), lambda i: (0, 0)),
        ],
        out_specs=pl.BlockSpec((_CBLK, HID), lambda i: (i, 0)),
        out_shape=jax.ShapeDtypeStruct((N_PAD, HID), jnp.float32),
    )(sums, cnt, w, b)


def _final_kernel(s_ref, c_ref, w_ref, b_ref, o_ref):
    s = s_ref[0] + s_ref[1]
    c = jnp.maximum(c_ref[0] + c_ref[1], 1.0)
    h = s / c[:, None]
    o_ref[...] = (
        jnp.dot(h, w_ref[...], preferred_element_type=jnp.float32)
        + b_ref[...]
    )


def _final(sums, cnt, w, b):
    return pl.pallas_call(
        _final_kernel,
        grid=(_CGRID,),
        in_specs=[
            pl.BlockSpec((N_CORES, _CBLK, HID), lambda i: (0, i, 0)),
            pl.BlockSpec((N_CORES, _CBLK), lambda i: (0, i)),
            pl.BlockSpec((HID, 2), lambda i: (0, 0)),
            pl.BlockSpec((1, 2), lambda i: (0, 0)),
        ],
        out_specs=pl.BlockSpec((_CBLK, 2), lambda i: (i, 0)),
        out_shape=jax.ShapeDtypeStruct((N_PAD, 2), jnp.float32),
    )(sums, cnt, w, b)


def kernel(features, src_t2e, dst_t2e, src_e2t, dst_e2t, entity_embed,
           W_t2e_0, b_t2e_0, W_e2t_0, b_e2t_0,
           W_t2e_1, b_t2e_1, W_e2t_1, b_e2t_1,
           W_out, b_out):
    wh = _mm1(features, W_t2e_0, b_t2e_0[None, :])
    sums1, cnt1 = _seg_sum(wh, src_t2e, dst_t2e)
    wh1 = _comb1(sums1, cnt1, W_e2t_1, b_e2t_1[None, :])
    sums2, cnt2 = _seg_sum(wh1, src_e2t, dst_e2t)
    out = _final(sums2, cnt2, W_out, b_out[None, :])
    return out[:N_NODES]


# speedup vs baseline: 35.6945x; 1.2628x over previous
import jax
import jax.numpy as jnp
from jax import lax
from jax.experimental import pallas as pl
from jax.experimental.pallas import tpu as pltpu
from jax.experimental.pallas import tpu_sc as plsc

N_NODES = 50000
N_PAD = 51200
N_EDGES = 1600000
HID = 16
N_CORES = 2
N_SUBCORES = 16
N_WORKERS = N_CORES * N_SUBCORES
E_PER_W = N_EDGES // N_WORKERS
CHUNK = 1000
N_CHUNKS = E_PER_W // CHUNK
UNROLL = 4
N_MAIN = 48
ROWS_PER_SUB = N_PAD // N_SUBCORES
IO_CHUNK = 400
N_IO = ROWS_PER_SUB // IO_CHUNK
ONES_BUF = 1024


def _make_seg_sum():
    mesh = plsc.VectorSubcoreMesh(
        core_axis_name="c", subcore_axis_name="s",
        num_cores=N_CORES, num_subcores=N_SUBCORES)

    def body(table, src, dst, sums_out, cnt_out,
             src_v, dst_v, rows_v, ones_v,
             acc_sh, cnt_sh,
             sem_l, sem_g, sem_s):
        cid = lax.axis_index("c")
        sid = lax.axis_index("s")
        wid = cid * N_SUBCORES + sid
        ebase = wid * E_PER_W

        zeros16 = jnp.zeros((16,), jnp.float32)
        ones16 = jnp.ones((16,), jnp.float32)

        def zrow(i, _):
            rows_v[0, i, :] = zeros16
            return 0
        lax.fori_loop(0, CHUNK, zrow, 0)

        def zrow1(i, _):
            ones_v[pl.ds(i * 16, 16)] = zeros16
            return 0
        lax.fori_loop(0, ONES_BUF // 16, zrow1, 0)

        rbase = sid * ROWS_PER_SUB
        for k in range(N_IO):
            pltpu.sync_copy(rows_v.at[0, pl.ds(0, IO_CHUNK)],
                            acc_sh.at[pl.ds(rbase + k * IO_CHUNK, IO_CHUNK)])
            pltpu.sync_copy(ones_v.at[pl.ds(0, IO_CHUNK)],
                            cnt_sh.at[pl.ds(rbase + k * IO_CHUNK, IO_CHUNK)])

        def orow(i, _):
            ones_v[pl.ds(i * 16, 16)] = ones16
            return 0
        lax.fori_loop(0, ONES_BUF // 16, orow, 0)
        plsc.subcore_barrier()

        def start_load(i, b):
            base = ebase + i * CHUNK
            pltpu.async_copy(src.at[pl.ds(base, CHUNK)], src_v.at[b],
                             sem_l.at[b])
            pltpu.async_copy(dst.at[pl.ds(base, CHUNK)], dst_v.at[b],
                             sem_l.at[b])

        def wait_load(i, b):
            base = ebase + i * CHUNK
            pltpu.make_async_copy(src.at[pl.ds(base, CHUNK)], src_v.at[b],
                                  sem_l.at[b]).wait()
            pltpu.make_async_copy(dst.at[pl.ds(base, CHUNK)], dst_v.at[b],
                                  sem_l.at[b]).wait()

        def start_scatter(b2, b4):
            pltpu.async_copy(rows_v.at[b2], acc_sh.at[dst_v.at[b4]],
                             sem_s.at[b2], add=True)
            pltpu.async_copy(ones_v.at[pl.ds(0, CHUNK)],
                             cnt_sh.at[dst_v.at[b4]],
                             sem_s.at[b2], add=True)

        def wait_scatter(b2, b4):
            pltpu.make_async_copy(rows_v.at[b2], acc_sh.at[dst_v.at[b4]],
                                  sem_s.at[b2]).wait()
            pltpu.make_async_copy(ones_v.at[pl.ds(0, CHUNK)],
                                  cnt_sh.at[dst_v.at[b4]],
                                  sem_s.at[b2]).wait()

        def gather(b2, b4):
            pltpu.async_copy(table.at[src_v.at[b4]], rows_v.at[b2],
                             sem_g).wait()

        start_load(0, 0)
        start_load(1, 1)

        def macro(m, _):
            for j in range(UNROLL):
                i = m * UNROLL + j
                j2 = j % 2
                j4 = j % 4

                @pl.when(i >= 2)
                def _():
                    wait_scatter(j2, (j4 + 2) % 4)

                start_load(i + 2, (j4 + 2) % 4)
                wait_load(i, j4)
                gather(j2, j4)
                start_scatter(j2, j4)
            return 0
        lax.fori_loop(0, N_MAIN // UNROLL, macro, 0)

        for i in (N_MAIN, N_MAIN + 1):
            j2, j4 = i % 2, i % 4
            wait_scatter(j2, (j4 + 2) % 4)
            wait_load(i, j4)
            gather(j2, j4)
            start_scatter(j2, j4)
        wait_scatter(N_MAIN % 2, N_MAIN % 4)
        wait_scatter((N_MAIN + 1) % 2, (N_MAIN + 1) % 4)
        plsc.subcore_barrier()

        for k in range(N_IO):
            off = rbase + k * IO_CHUNK
            pltpu.sync_copy(acc_sh.at[pl.ds(off, IO_CHUNK)],
                            rows_v.at[0, pl.ds(0, IO_CHUNK)])
            pltpu.sync_copy(rows_v.at[0, pl.ds(0, IO_CHUNK)],
                            sums_out.at[cid, pl.ds(off, IO_CHUNK)])
            pltpu.sync_copy(cnt_sh.at[pl.ds(off, IO_CHUNK)],
                            ones_v.at[pl.ds(0, IO_CHUNK)])
            pltpu.sync_copy(ones_v.at[pl.ds(0, IO_CHUNK)],
                            cnt_out.at[cid, pl.ds(off, IO_CHUNK)])

    return pl.kernel(
        body,
        out_type=[
            jax.ShapeDtypeStruct((N_CORES, N_PAD, HID), jnp.float32),
            jax.ShapeDtypeStruct((N_CORES, N_PAD), jnp.float32),
        ],
        mesh=mesh,
        compiler_params=pltpu.CompilerParams(use_tc_tiling_on_sc=False),
        scratch_types=[
            pltpu.VMEM((UNROLL, CHUNK), jnp.int32),
            pltpu.VMEM((UNROLL, CHUNK), jnp.int32),
            pltpu.VMEM((2, CHUNK, HID), jnp.float32),
            pltpu.VMEM((ONES_BUF,), jnp.float32),
            pltpu.VMEM_SHARED((N_PAD, HID), jnp.float32),
            pltpu.VMEM_SHARED((N_PAD,), jnp.float32),
            pltpu.SemaphoreType.DMA((UNROLL,)),
            pltpu.SemaphoreType.DMA,
            pltpu.SemaphoreType.DMA((2,)),
        ],
    )


_seg_sum = _make_seg_sum()


def _mm1_kernel(x_ref, w_ref, b_ref, o_ref):
    o_ref[...] = (
        jnp.dot(x_ref[...], w_ref[...], preferred_element_type=jnp.float32)
        + b_ref[...]
    )


_MM1_BLOCK = 2000


def _mm1(features, w, b):
    grid = N_NODES // _MM1_BLOCK
    return pl.pallas_call(
        _mm1_kernel,
        grid=(grid,),
        in_specs=[
            pl.BlockSpec((_MM1_BLOCK, 128), lambda i: (i, 0)),
            pl.BlockSpec((128, HID), lambda i: (0, 0)),
            pl.BlockSpec((1, HID), lambda i: (0, 0)),
        ],
        out_specs=pl.BlockSpec((_MM1_BLOCK, HID), lambda i: (i, 0)),
        out_shape=jax.ShapeDtypeStruct((N_NODES, HID), jnp.float32),
    )(features, w, b)


_CBLK = 3200
_CGRID = N_PAD // _CBLK


def _comb1_kernel(s_ref, c_ref, w_ref, b_ref, o_ref):
    s = s_ref[0] + s_ref[1]
    c = jnp.maximum(c_ref[0] + c_ref[1], 1.0)
    h = s / c[:, None]
    h = jnp.where(h >= 0, h, 0.01 * h)
    o_ref[...] = (
        jnp.dot(h, w_ref[...], preferred_element_type=jnp.float32)
        + b_ref[...]
    )


def _comb1(sums, cnt, w, b):
    return pl.pallas_call(
        _comb1_kernel,
        grid=(_CGRID,),
        in_specs=[
            pl.BlockSpec((N_CORES, _CBLK, HID), lambda i: (0, i, 0)),
            pl.BlockSpec((N_CORES, _CBLK), lambda i: (0, i)),
            pl.BlockSpec((HID, HID), lambda i: (0, 0)),
            pl.BlockSpec((1, HID), lambda i: (0, 0)),
        ],
        out_specs=pl.BlockSpec((_CBLK, HID), lambda i: (i, 0)),
        out_shape=jax.ShapeDtypeStruct((N_PAD, HID), jnp.float32),
    )(sums, cnt, w, b)


def _final_kernel(s_ref, c_ref, w_ref, b_ref, o_ref):
    s = s_ref[0] + s_ref[1]
    c = jnp.maximum(c_ref[0] + c_ref[1], 1.0)
    h = s / c[:, None]
    o_ref[...] = (
        jnp.dot(h, w_ref[...], preferred_element_type=jnp.float32)
        + b_ref[...]
    )


def _final(sums, cnt, w, b):
    return pl.pallas_call(
        _final_kernel,
        grid=(_CGRID,),
        in_specs=[
            pl.BlockSpec((N_CORES, _CBLK, HID), lambda i: (0, i, 0)),
            pl.BlockSpec((N_CORES, _CBLK), lambda i: (0, i)),
            pl.BlockSpec((HID, 2), lambda i: (0, 0)),
            pl.BlockSpec((1, 2), lambda i: (0, 0)),
        ],
        out_specs=pl.BlockSpec((_CBLK, 2), lambda i: (i, 0)),
        out_shape=jax.ShapeDtypeStruct((N_PAD, 2), jnp.float32),
    )(sums, cnt, w, b)


def kernel(features, src_t2e, dst_t2e, src_e2t, dst_e2t, entity_embed,
           W_t2e_0, b_t2e_0, W_e2t_0, b_e2t_0,
           W_t2e_1, b_t2e_1, W_e2t_1, b_e2t_1,
           W_out, b_out):
    wh = _mm1(features, W_t2e_0, b_t2e_0[None, :])
    sums1, cnt1 = _seg_sum(wh, src_t2e, dst_t2e)
    wh1 = _comb1(sums1, cnt1, W_e2t_1, b_e2t_1[None, :])
    sums2, cnt2 = _seg_sum(wh1, src_e2t, dst_e2t)
    out = _final(sums2, cnt2, W_out, b_out[None, :])
    return out[:N_NODES]


# speedup vs baseline: 44.2257x; 1.2390x over previous
import jax
import jax.numpy as jnp
from jax import lax
from jax.experimental import pallas as pl
from jax.experimental.pallas import tpu as pltpu
from jax.experimental.pallas import tpu_sc as plsc

N_NODES = 50000
N_PAD = 51200
N_EDGES = 1600000
HID = 16
N_CORES = 2
N_SUBCORES = 16
N_WORKERS = N_CORES * N_SUBCORES
E_PER_W = N_EDGES // N_WORKERS
CHUNK = 1000
N_CHUNKS = E_PER_W // CHUNK
UNROLL = 4
N_MAIN = 48
ROWS_PER_SUB = N_PAD // N_SUBCORES
IO_CHUNK = 400
N_IO = ROWS_PER_SUB // IO_CHUNK
ONES_BUF = 1024


def _make_seg_sum():
    mesh = plsc.VectorSubcoreMesh(
        core_axis_name="c", subcore_axis_name="s",
        num_cores=N_CORES, num_subcores=N_SUBCORES)

    def body(table, src, dst, sums_out, cnt_out,
             src_v, dst_v, rows_v, ones_v,
             acc_sh, cnt_sh,
             sem_l, sem_g, sem_s):
        cid = lax.axis_index("c")
        sid = lax.axis_index("s")
        wid = cid * N_SUBCORES + sid
        ebase = wid * E_PER_W

        zeros16 = jnp.zeros((16,), jnp.float32)
        ones16 = jnp.ones((16,), jnp.float32)

        def zrow(i, _):
            rows_v[0, i, :] = zeros16
            return 0
        lax.fori_loop(0, CHUNK, zrow, 0)

        def zrow1(i, _):
            ones_v[pl.ds(i * 16, 16)] = zeros16
            return 0
        lax.fori_loop(0, ONES_BUF // 16, zrow1, 0)

        rbase = sid * ROWS_PER_SUB
        for k in range(N_IO):
            pltpu.sync_copy(rows_v.at[0, pl.ds(0, IO_CHUNK)],
                            acc_sh.at[pl.ds(rbase + k * IO_CHUNK, IO_CHUNK)])
            pltpu.sync_copy(ones_v.at[pl.ds(0, IO_CHUNK)],
                            cnt_sh.at[pl.ds(rbase + k * IO_CHUNK, IO_CHUNK)])

        def orow(i, _):
            ones_v[pl.ds(i * 16, 16)] = ones16
            return 0
        lax.fori_loop(0, ONES_BUF // 16, orow, 0)
        plsc.subcore_barrier()

        def start_load(i, b):
            base = ebase + i * CHUNK
            pltpu.async_copy(src.at[pl.ds(base, CHUNK)], src_v.at[b],
                             sem_l.at[b])
            pltpu.async_copy(dst.at[pl.ds(base, CHUNK)], dst_v.at[b],
                             sem_l.at[b])

        def wait_load(i, b):
            base = ebase + i * CHUNK
            pltpu.make_async_copy(src.at[pl.ds(base, CHUNK)], src_v.at[b],
                                  sem_l.at[b]).wait()
            pltpu.make_async_copy(dst.at[pl.ds(base, CHUNK)], dst_v.at[b],
                                  sem_l.at[b]).wait()

        def start_scatter(b2, b4):
            pltpu.async_copy(rows_v.at[b2], acc_sh.at[dst_v.at[b4]],
                             sem_s.at[b2], add=True)
            pltpu.async_copy(ones_v.at[pl.ds(0, CHUNK)],
                             cnt_sh.at[dst_v.at[b4]],
                             sem_s.at[b2], add=True)

        def wait_scatter(b2, b4):
            pltpu.make_async_copy(rows_v.at[b2], acc_sh.at[dst_v.at[b4]],
                                  sem_s.at[b2]).wait()
            pltpu.make_async_copy(ones_v.at[pl.ds(0, CHUNK)],
                                  cnt_sh.at[dst_v.at[b4]],
                                  sem_s.at[b2]).wait()

        def gather(b2, b4):
            pltpu.async_copy(table.at[src_v.at[b4]], rows_v.at[b2],
                             sem_g).wait()

        start_load(0, 0)
        start_load(1, 1)

        def macro(m, _):
            for j in range(UNROLL):
                i = m * UNROLL + j
                j2 = j % 2
                j4 = j % 4

                @pl.when(i >= 2)
                def _():
                    wait_scatter(j2, (j4 + 2) % 4)

                start_load(i + 2, (j4 + 2) % 4)
                wait_load(i, j4)
                gather(j2, j4)
                start_scatter(j2, j4)
            return 0
        lax.fori_loop(0, N_MAIN // UNROLL, macro, 0)

        for i in (N_MAIN, N_MAIN + 1):
            j2, j4 = i % 2, i % 4
            wait_scatter(j2, (j4 + 2) % 4)
            wait_load(i, j4)
            gather(j2, j4)
            start_scatter(j2, j4)
        wait_scatter(N_MAIN % 2, N_MAIN % 4)
        wait_scatter((N_MAIN + 1) % 2, (N_MAIN + 1) % 4)
        plsc.subcore_barrier()

        for k in range(N_IO):
            off = rbase + k * IO_CHUNK
            pltpu.sync_copy(acc_sh.at[pl.ds(off, IO_CHUNK)],
                            rows_v.at[0, pl.ds(0, IO_CHUNK)])
            pltpu.sync_copy(rows_v.at[0, pl.ds(0, IO_CHUNK)],
                            sums_out.at[cid, pl.ds(off, IO_CHUNK)])
            pltpu.sync_copy(cnt_sh.at[pl.ds(off, IO_CHUNK)],
                            ones_v.at[pl.ds(0, IO_CHUNK)])
            pltpu.sync_copy(ones_v.at[pl.ds(0, IO_CHUNK)],
                            cnt_out.at[cid, pl.ds(off, IO_CHUNK)])

    return pl.kernel(
        body,
        out_type=[
            jax.ShapeDtypeStruct((N_CORES, N_PAD, HID), jnp.float32),
            jax.ShapeDtypeStruct((N_CORES, N_PAD), jnp.float32),
        ],
        mesh=mesh,
        compiler_params=pltpu.CompilerParams(use_tc_tiling_on_sc=False),
        scratch_types=[
            pltpu.VMEM((UNROLL, CHUNK), jnp.int32),
            pltpu.VMEM((UNROLL, CHUNK), jnp.int32),
            pltpu.VMEM((2, CHUNK, HID), jnp.float32),
            pltpu.VMEM((ONES_BUF,), jnp.float32),
            pltpu.VMEM_SHARED((N_PAD, HID), jnp.float32),
            pltpu.VMEM_SHARED((N_PAD,), jnp.float32),
            pltpu.SemaphoreType.DMA((UNROLL,)),
            pltpu.SemaphoreType.DMA,
            pltpu.SemaphoreType.DMA((2,)),
        ],
    )


_seg_sum = _make_seg_sum()


_MM1_BLOCK = 2048


def _mm1_kernel(x_ref, w_ref, b_ref, o_ref):
    y = (
        jnp.dot(x_ref[...], w_ref[...], preferred_element_type=jnp.float32)
        + b_ref[...]
    )
    o_ref[...] = y.reshape(_MM1_BLOCK // 8, 8, HID)


def _mm1(features, w, b):
    grid = N_PAD // _MM1_BLOCK
    return pl.pallas_call(
        _mm1_kernel,
        grid=(grid,),
        in_specs=[
            pl.BlockSpec((_MM1_BLOCK, 128), lambda i: (i, 0)),
            pl.BlockSpec((128, HID), lambda i: (0, 0)),
            pl.BlockSpec((1, HID), lambda i: (0, 0)),
        ],
        out_specs=pl.BlockSpec((_MM1_BLOCK // 8, 8, HID), lambda i: (i, 0, 0)),
        out_shape=jax.ShapeDtypeStruct((N_PAD // 8, 8, HID), jnp.float32),
    )(features, w, b)


_CBLK = 3200


def _comb1_kernel(s_ref, c_ref, w_ref, b_ref, o_ref):
    s = s_ref[0] + s_ref[1]
    c = jnp.maximum(c_ref[0] + c_ref[1], 1.0)
    cb = jnp.repeat(c, HID, axis=1)
    h = s / cb
    h = jnp.where(h >= 0, h, 0.01 * h)
    o_ref[...] = (
        jnp.dot(h, w_ref[...], preferred_element_type=jnp.float32)
        + b_ref[...]
    )


def _comb1(sums128, cnt8, w_bd, b128):
    return pl.pallas_call(
        _comb1_kernel,
        grid=(N_PAD // 8 // _CBLK,),
        in_specs=[
            pl.BlockSpec((N_CORES, _CBLK, 128), lambda i: (0, i, 0)),
            pl.BlockSpec((N_CORES, _CBLK, 8), lambda i: (0, i, 0)),
            pl.BlockSpec((128, 128), lambda i: (0, 0)),
            pl.BlockSpec((1, 128), lambda i: (0, 0)),
        ],
        out_specs=pl.BlockSpec((_CBLK, 128), lambda i: (i, 0)),
        out_shape=jax.ShapeDtypeStruct((N_PAD // 8, 128), jnp.float32),
    )(sums128, cnt8, w_bd, b128)


_FROWS = N_PAD // 16
_FOUT = N_NODES // 16


def _final_kernel(s_ref, c_ref, w_ref, b_ref, o_ref):
    s = s_ref[0] + s_ref[1]
    c = jnp.maximum(c_ref[0] + c_ref[1], 1.0)
    cb = jnp.repeat(c, HID, axis=1)
    h = s / cb
    y = (
        jnp.dot(h, w_ref[...], preferred_element_type=jnp.float32)
        + b_ref[...]
    )
    o_ref[...] = y[:_FOUT, :]


def _final(sums256, cnt16, w_bd, b32):
    return pl.pallas_call(
        _final_kernel,
        in_specs=[
            pl.BlockSpec((N_CORES, _FROWS, 256), lambda: (0, 0, 0)),
            pl.BlockSpec((N_CORES, _FROWS, HID), lambda: (0, 0, 0)),
            pl.BlockSpec((256, 32), lambda: (0, 0)),
            pl.BlockSpec((1, 32), lambda: (0, 0)),
        ],
        out_specs=pl.BlockSpec((_FOUT, 32), lambda: (0, 0)),
        out_shape=jax.ShapeDtypeStruct((_FOUT, 32), jnp.float32),
    )(sums256, cnt16, w_bd, b32)


def kernel(features, src_t2e, dst_t2e, src_e2t, dst_e2t, entity_embed,
           W_t2e_0, b_t2e_0, W_e2t_0, b_e2t_0,
           W_t2e_1, b_t2e_1, W_e2t_1, b_e2t_1,
           W_out, b_out):
    w1_bd = jnp.kron(jnp.eye(8, dtype=jnp.float32), W_e2t_1)
    b1_128 = jnp.tile(b_e2t_1, 8)[None, :]
    wout_bd = jnp.kron(jnp.eye(16, dtype=jnp.float32), W_out)
    bout_32 = jnp.tile(b_out, 16)[None, :]

    wh = _mm1(features, W_t2e_0, b_t2e_0[None, :]).reshape(N_PAD, HID)
    sums1, cnt1 = _seg_sum(wh, src_t2e, dst_t2e)
    wh1_128 = _comb1(sums1.reshape(N_CORES, N_PAD // 8, 128),
                     cnt1.reshape(N_CORES, N_PAD // 8, 8),
                     w1_bd, b1_128)
    wh1 = wh1_128.reshape(N_PAD, HID)
    sums2, cnt2 = _seg_sum(wh1, src_e2t, dst_e2t)
    out32 = _final(sums2.reshape(N_CORES, N_PAD // 16, 256),
                   cnt2.reshape(N_CORES, N_PAD // 16, HID),
                   wout_bd, bout_32)
    return out32.reshape(N_NODES, 2)


# speedup vs baseline: 44.9114x; 1.0155x over previous
import jax
import jax.numpy as jnp
from jax import lax
from jax.experimental import pallas as pl
from jax.experimental.pallas import tpu as pltpu
from jax.experimental.pallas import tpu_sc as plsc

N_NODES = 50000
N_PAD = 51200
N_EDGES = 1600000
HID = 16
N_CORES = 2
N_SUBCORES = 16
N_WORKERS = N_CORES * N_SUBCORES
E_PER_W = N_EDGES // N_WORKERS
CHUNK = 1000
N_CHUNKS = E_PER_W // CHUNK
UNROLL = 4
N_MAIN = 48
ROWS_PER_SUB = N_PAD // N_SUBCORES
IO_CHUNK = 400
N_IO = ROWS_PER_SUB // IO_CHUNK
ONES_BUF = 1024


def _make_seg_sum():
    mesh = plsc.VectorSubcoreMesh(
        core_axis_name="c", subcore_axis_name="s",
        num_cores=N_CORES, num_subcores=N_SUBCORES)

    def body(table, src, dst, sums_out, cnt_out,
             src_v, dst_v, rows_v, ones_v,
             acc_sh, cnt_sh,
             sem_l, sem_g, sem_s):
        cid = lax.axis_index("c")
        sid = lax.axis_index("s")
        wid = cid * N_SUBCORES + sid
        ebase = wid * E_PER_W

        zeros16 = jnp.zeros((16,), jnp.float32)
        ones16 = jnp.ones((16,), jnp.float32)

        def zrow(i, _):
            rows_v[0, i, :] = zeros16
            return 0
        lax.fori_loop(0, CHUNK, zrow, 0)

        def zrow1(i, _):
            ones_v[pl.ds(i * 16, 16)] = zeros16
            return 0
        lax.fori_loop(0, ONES_BUF // 16, zrow1, 0)

        rbase = sid * ROWS_PER_SUB
        for k in range(N_IO):
            pltpu.sync_copy(rows_v.at[0, pl.ds(0, IO_CHUNK)],
                            acc_sh.at[pl.ds(rbase + k * IO_CHUNK, IO_CHUNK)])
            pltpu.sync_copy(ones_v.at[pl.ds(0, IO_CHUNK)],
                            cnt_sh.at[pl.ds(rbase + k * IO_CHUNK, IO_CHUNK)])

        def orow(i, _):
            ones_v[pl.ds(i * 16, 16)] = ones16
            return 0
        lax.fori_loop(0, ONES_BUF // 16, orow, 0)
        plsc.subcore_barrier()

        def start_load(i, b):
            base = ebase + i * CHUNK
            pltpu.async_copy(src.at[pl.ds(base, CHUNK)], src_v.at[b],
                             sem_l.at[b])
            pltpu.async_copy(dst.at[pl.ds(base, CHUNK)], dst_v.at[b],
                             sem_l.at[b])

        def wait_load(i, b):
            base = ebase + i * CHUNK
            pltpu.make_async_copy(src.at[pl.ds(base, CHUNK)], src_v.at[b],
                                  sem_l.at[b]).wait()
            pltpu.make_async_copy(dst.at[pl.ds(base, CHUNK)], dst_v.at[b],
                                  sem_l.at[b]).wait()

        def start_scatter(b2, b4):
            pltpu.async_copy(rows_v.at[b2], acc_sh.at[dst_v.at[b4]],
                             sem_s.at[b2], add=True)
            pltpu.async_copy(ones_v.at[pl.ds(0, CHUNK)],
                             cnt_sh.at[dst_v.at[b4]],
                             sem_s.at[b2], add=True)

        def wait_scatter(b2, b4):
            pltpu.make_async_copy(rows_v.at[b2], acc_sh.at[dst_v.at[b4]],
                                  sem_s.at[b2]).wait()
            pltpu.make_async_copy(ones_v.at[pl.ds(0, CHUNK)],
                                  cnt_sh.at[dst_v.at[b4]],
                                  sem_s.at[b2]).wait()

        def gather(b2, b4):
            pltpu.async_copy(table.at[src_v.at[b4]], rows_v.at[b2],
                             sem_g).wait()

        start_load(0, 0)
        start_load(1, 1)

        def macro(m, _):
            for j in range(UNROLL):
                i = m * UNROLL + j
                j2 = j % 2
                j4 = j % 4

                @pl.when(i >= 2)
                def _():
                    wait_scatter(j2, (j4 + 2) % 4)

                start_load(i + 2, (j4 + 2) % 4)
                wait_load(i, j4)
                gather(j2, j4)
                start_scatter(j2, j4)
            return 0
        lax.fori_loop(0, N_MAIN // UNROLL, macro, 0)

        for i in (N_MAIN, N_MAIN + 1):
            j2, j4 = i % 2, i % 4
            wait_scatter(j2, (j4 + 2) % 4)
            wait_load(i, j4)
            gather(j2, j4)
            start_scatter(j2, j4)
        wait_scatter(N_MAIN % 2, N_MAIN % 4)
        wait_scatter((N_MAIN + 1) % 2, (N_MAIN + 1) % 4)
        plsc.subcore_barrier()

        for k in range(N_IO):
            off = rbase + k * IO_CHUNK
            pltpu.sync_copy(acc_sh.at[pl.ds(off, IO_CHUNK)],
                            rows_v.at[0, pl.ds(0, IO_CHUNK)])
            pltpu.sync_copy(rows_v.at[0, pl.ds(0, IO_CHUNK)],
                            sums_out.at[cid, pl.ds(off, IO_CHUNK)])
            pltpu.sync_copy(cnt_sh.at[pl.ds(off, IO_CHUNK)],
                            ones_v.at[pl.ds(0, IO_CHUNK)])
            pltpu.sync_copy(ones_v.at[pl.ds(0, IO_CHUNK)],
                            cnt_out.at[cid, pl.ds(off, IO_CHUNK)])

    return pl.kernel(
        body,
        out_type=[
            jax.ShapeDtypeStruct((N_CORES, N_PAD, HID), jnp.float32),
            jax.ShapeDtypeStruct((N_CORES, N_PAD), jnp.float32),
        ],
        mesh=mesh,
        compiler_params=pltpu.CompilerParams(use_tc_tiling_on_sc=False),
        scratch_types=[
            pltpu.VMEM((UNROLL, CHUNK), jnp.int32),
            pltpu.VMEM((UNROLL, CHUNK), jnp.int32),
            pltpu.VMEM((2, CHUNK, HID), jnp.float32),
            pltpu.VMEM((ONES_BUF,), jnp.float32),
            pltpu.VMEM_SHARED((N_PAD, HID), jnp.float32),
            pltpu.VMEM_SHARED((N_PAD,), jnp.float32),
            pltpu.SemaphoreType.DMA((UNROLL,)),
            pltpu.SemaphoreType.DMA,
            pltpu.SemaphoreType.DMA((2,)),
        ],
    )


_seg_sum = _make_seg_sum()


_MM1_ROWS = N_NODES // 8
_MM1_KBLK = 256
_MM1_KGRID = 1024 // _MM1_KBLK


def _mm1_kernel(x_ref, w_ref, b_ref, o_ref):
    j = pl.program_id(0)

    @pl.when(j == 0)
    def _():
        o_ref[...] = jnp.broadcast_to(b_ref[...], (_MM1_ROWS, 128))

    o_ref[...] += jnp.dot(x_ref[...], w_ref[...],
                          preferred_element_type=jnp.float32)


def _mm1(features8, w_bd, b128):
    return pl.pallas_call(
        _mm1_kernel,
        grid=(_MM1_KGRID,),
        in_specs=[
            pl.BlockSpec((_MM1_ROWS, _MM1_KBLK), lambda j: (0, j)),
            pl.BlockSpec((_MM1_KBLK, 128), lambda j: (j, 0)),
            pl.BlockSpec((1, 128), lambda j: (0, 0)),
        ],
        out_specs=pl.BlockSpec((_MM1_ROWS, 128), lambda j: (0, 0)),
        out_shape=jax.ShapeDtypeStruct((_MM1_ROWS, 128), jnp.float32),
    )(features8, w_bd, b128)


_CBLK = 3200


def _comb1_kernel(s_ref, c_ref, w_ref, b_ref, o_ref):
    s = s_ref[0] + s_ref[1]
    c = jnp.maximum(c_ref[0] + c_ref[1], 1.0)
    cb = jnp.repeat(c, HID, axis=1)
    h = s / cb
    h = jnp.where(h >= 0, h, 0.01 * h)
    o_ref[...] = (
        jnp.dot(h, w_ref[...], preferred_element_type=jnp.float32)
        + b_ref[...]
    )


def _comb1(sums128, cnt8, w_bd, b128):
    return pl.pallas_call(
        _comb1_kernel,
        grid=(N_PAD // 8 // _CBLK,),
        in_specs=[
            pl.BlockSpec((N_CORES, _CBLK, 128), lambda i: (0, i, 0)),
            pl.BlockSpec((N_CORES, _CBLK, 8), lambda i: (0, i, 0)),
            pl.BlockSpec((128, 128), lambda i: (0, 0)),
            pl.BlockSpec((1, 128), lambda i: (0, 0)),
        ],
        out_specs=pl.BlockSpec((_CBLK, 128), lambda i: (i, 0)),
        out_shape=jax.ShapeDtypeStruct((N_PAD // 8, 128), jnp.float32),
    )(sums128, cnt8, w_bd, b128)


_FROWS = N_PAD // 16
_FOUT = N_NODES // 16


def _final_kernel(s_ref, c_ref, w_ref, b_ref, o_ref):
    s = s_ref[0] + s_ref[1]
    c = jnp.maximum(c_ref[0] + c_ref[1], 1.0)
    cb = jnp.repeat(c, HID, axis=1)
    h = s / cb
    y = (
        jnp.dot(h, w_ref[...], preferred_element_type=jnp.float32)
        + b_ref[...]
    )
    o_ref[...] = y[:_FOUT, :]


def _final(sums256, cnt16, w_bd, b32):
    return pl.pallas_call(
        _final_kernel,
        in_specs=[
            pl.BlockSpec((N_CORES, _FROWS, 256), lambda: (0, 0, 0)),
            pl.BlockSpec((N_CORES, _FROWS, HID), lambda: (0, 0, 0)),
            pl.BlockSpec((256, 32), lambda: (0, 0)),
            pl.BlockSpec((1, 32), lambda: (0, 0)),
        ],
        out_specs=pl.BlockSpec((_FOUT, 32), lambda: (0, 0)),
        out_shape=jax.ShapeDtypeStruct((_FOUT, 32), jnp.float32),
    )(sums256, cnt16, w_bd, b32)


def kernel(features, src_t2e, dst_t2e, src_e2t, dst_e2t, entity_embed,
           W_t2e_0, b_t2e_0, W_e2t_0, b_e2t_0,
           W_t2e_1, b_t2e_1, W_e2t_1, b_e2t_1,
           W_out, b_out):
    w1_bd = jnp.kron(jnp.eye(8, dtype=jnp.float32), W_e2t_1)
    b1_128 = jnp.tile(b_e2t_1, 8)[None, :]
    wout_bd = jnp.kron(jnp.eye(16, dtype=jnp.float32), W_out)
    bout_32 = jnp.tile(b_out, 16)[None, :]
    w0_bd = jnp.kron(jnp.eye(8, dtype=jnp.float32), W_t2e_0)
    b0_128 = jnp.tile(b_t2e_0, 8)[None, :]

    wh = _mm1(features.reshape(_MM1_ROWS, 1024), w0_bd,
              b0_128).reshape(N_NODES, HID)
    sums1, cnt1 = _seg_sum(wh, src_t2e, dst_t2e)
    wh1_128 = _comb1(sums1.reshape(N_CORES, N_PAD // 8, 128),
                     cnt1.reshape(N_CORES, N_PAD // 8, 8),
                     w1_bd, b1_128)
    wh1 = wh1_128.reshape(N_PAD, HID)
    sums2, cnt2 = _seg_sum(wh1, src_e2t, dst_e2t)
    out32 = _final(sums2.reshape(N_CORES, N_PAD // 16, 256),
                   cnt2.reshape(N_CORES, N_PAD // 16, HID),
                   wout_bd, bout_32)
    return out32.reshape(N_NODES, 2)


# speedup vs baseline: 52.7996x; 1.1756x over previous
import jax
import jax.numpy as jnp
from jax import lax
from jax.experimental import pallas as pl
from jax.experimental.pallas import tpu as pltpu
from jax.experimental.pallas import tpu_sc as plsc

N_NODES = 50000
N_PAD = 51200
N_EDGES = 1600000
HID = 16
N_CORES = 2
N_SUBCORES = 16
N_WORKERS = N_CORES * N_SUBCORES
E_PER_W = N_EDGES // N_WORKERS
CHUNK = 1000
N_CHUNKS = E_PER_W // CHUNK
UNROLL = 4
N_MAIN = 48
ROWS_PER_SUB = N_PAD // N_SUBCORES
IO_CHUNK = 400
N_IO = ROWS_PER_SUB // IO_CHUNK
ONES_BUF = 1024


def _make_seg_sum():
    mesh = plsc.VectorSubcoreMesh(
        core_axis_name="c", subcore_axis_name="s",
        num_cores=N_CORES, num_subcores=N_SUBCORES)

    def body(table, src, dst, sums_out, cnt_out,
             src_v, dst_v, rows_v, ones_v,
             acc_sh, cnt_sh,
             sem_l, sem_g, sem_s):
        cid = lax.axis_index("c")
        sid = lax.axis_index("s")
        wid = cid * N_SUBCORES + sid
        ebase = wid * E_PER_W

        zeros16 = jnp.zeros((16,), jnp.float32)
        ones16 = jnp.ones((16,), jnp.float32)

        def zrow(i, _):
            rows_v[0, i, :] = zeros16
            return 0
        lax.fori_loop(0, CHUNK, zrow, 0)

        def zrow1(i, _):
            ones_v[pl.ds(i * 16, 16)] = zeros16
            return 0
        lax.fori_loop(0, ONES_BUF // 16, zrow1, 0)

        rbase = sid * ROWS_PER_SUB
        for k in range(N_IO):
            pltpu.sync_copy(rows_v.at[0, pl.ds(0, IO_CHUNK)],
                            acc_sh.at[pl.ds(rbase + k * IO_CHUNK, IO_CHUNK)])
            pltpu.sync_copy(ones_v.at[pl.ds(0, IO_CHUNK)],
                            cnt_sh.at[pl.ds(rbase + k * IO_CHUNK, IO_CHUNK)])

        def orow(i, _):
            ones_v[pl.ds(i * 16, 16)] = ones16
            return 0
        lax.fori_loop(0, ONES_BUF // 16, orow, 0)
        plsc.subcore_barrier()

        def start_load(i, b):
            base = ebase + i * CHUNK
            pltpu.async_copy(src.at[pl.ds(base, CHUNK)], src_v.at[b],
                             sem_l.at[b])
            pltpu.async_copy(dst.at[pl.ds(base, CHUNK)], dst_v.at[b],
                             sem_l.at[b])

        def wait_load(i, b):
            base = ebase + i * CHUNK
            pltpu.make_async_copy(src.at[pl.ds(base, CHUNK)], src_v.at[b],
                                  sem_l.at[b]).wait()
            pltpu.make_async_copy(dst.at[pl.ds(base, CHUNK)], dst_v.at[b],
                                  sem_l.at[b]).wait()

        def start_scatter(b):
            pltpu.async_copy(rows_v.at[b], acc_sh.at[dst_v.at[b]],
                             sem_s.at[b], add=True)
            pltpu.async_copy(ones_v.at[pl.ds(0, CHUNK)],
                             cnt_sh.at[dst_v.at[b]],
                             sem_s.at[b], add=True)

        def wait_scatter(b):
            pltpu.make_async_copy(rows_v.at[b], acc_sh.at[dst_v.at[b]],
                                  sem_s.at[b]).wait()
            pltpu.make_async_copy(ones_v.at[pl.ds(0, CHUNK)],
                                  cnt_sh.at[dst_v.at[b]],
                                  sem_s.at[b]).wait()

        def start_gather(b):
            pltpu.async_copy(table.at[src_v.at[b]], rows_v.at[b],
                             sem_g.at[b])

        def wait_gather(b):
            pltpu.make_async_copy(table.at[src_v.at[b]], rows_v.at[b],
                                  sem_g.at[b]).wait()

        start_load(0, 0)
        start_load(1, 1)
        wait_load(0, 0)
        start_gather(0)

        def macro(m, _):
            for j in range(UNROLL):
                i = m * UNROLL + j

                @pl.when(i >= 2)
                def _():
                    wait_scatter((j + 2) % 4)

                start_load(i + 2, (j + 2) % 4)
                wait_load(i + 1, (j + 1) % 4)
                start_gather((j + 1) % 4)
                wait_gather(j)
                start_scatter(j)
            return 0
        lax.fori_loop(0, N_MAIN // UNROLL, macro, 0)

        wait_load(N_MAIN + 1, 1)
        start_gather(1)
        wait_gather(0)
        start_scatter(0)
        wait_gather(1)
        start_scatter(1)
        for b in (2, 3, 0, 1):
            wait_scatter(b)
        plsc.subcore_barrier()

        for k in range(N_IO):
            off = rbase + k * IO_CHUNK
            pltpu.sync_copy(acc_sh.at[pl.ds(off, IO_CHUNK)],
                            rows_v.at[0, pl.ds(0, IO_CHUNK)])
            pltpu.sync_copy(rows_v.at[0, pl.ds(0, IO_CHUNK)],
                            sums_out.at[cid, pl.ds(off, IO_CHUNK)])
            pltpu.sync_copy(cnt_sh.at[pl.ds(off, IO_CHUNK)],
                            ones_v.at[pl.ds(0, IO_CHUNK)])
            pltpu.sync_copy(ones_v.at[pl.ds(0, IO_CHUNK)],
                            cnt_out.at[cid, pl.ds(off, IO_CHUNK)])

    return pl.kernel(
        body,
        out_type=[
            jax.ShapeDtypeStruct((N_CORES, N_PAD, HID), jnp.float32),
            jax.ShapeDtypeStruct((N_CORES, N_PAD), jnp.float32),
        ],
        mesh=mesh,
        compiler_params=pltpu.CompilerParams(use_tc_tiling_on_sc=False),
        scratch_types=[
            pltpu.VMEM((UNROLL, CHUNK), jnp.int32),
            pltpu.VMEM((UNROLL, CHUNK), jnp.int32),
            pltpu.VMEM((UNROLL, CHUNK, HID), jnp.float32),
            pltpu.VMEM((ONES_BUF,), jnp.float32),
            pltpu.VMEM_SHARED((N_PAD, HID), jnp.float32),
            pltpu.VMEM_SHARED((N_PAD,), jnp.float32),
            pltpu.SemaphoreType.DMA((UNROLL,)),
            pltpu.SemaphoreType.DMA((UNROLL,)),
            pltpu.SemaphoreType.DMA((UNROLL,)),
        ],
    )


_seg_sum = _make_seg_sum()


_MM1_ROWS = N_NODES // 8
_MM1_KBLK = 256
_MM1_KGRID = 1024 // _MM1_KBLK


def _mm1_kernel(x_ref, w_ref, b_ref, o_ref):
    j = pl.program_id(0)

    @pl.when(j == 0)
    def _():
        o_ref[...] = jnp.broadcast_to(b_ref[...], (_MM1_ROWS, 128))

    o_ref[...] += jnp.dot(x_ref[...], w_ref[...],
                          preferred_element_type=jnp.float32)


def _mm1(features8, w_bd, b128):
    return pl.pallas_call(
        _mm1_kernel,
        grid=(_MM1_KGRID,),
        in_specs=[
            pl.BlockSpec((_MM1_ROWS, _MM1_KBLK), lambda j: (0, j)),
            pl.BlockSpec((_MM1_KBLK, 128), lambda j: (j, 0)),
            pl.BlockSpec((1, 128), lambda j: (0, 0)),
        ],
        out_specs=pl.BlockSpec((_MM1_ROWS, 128), lambda j: (0, 0)),
        out_shape=jax.ShapeDtypeStruct((_MM1_ROWS, 128), jnp.float32),
    )(features8, w_bd, b128)


_CBLK = 3200


def _comb1_kernel(s_ref, c_ref, w_ref, b_ref, o_ref):
    s = s_ref[0] + s_ref[1]
    c = jnp.maximum(c_ref[0] + c_ref[1], 1.0)
    cb = jnp.repeat(c, HID, axis=1)
    h = s / cb
    h = jnp.where(h >= 0, h, 0.01 * h)
    o_ref[...] = (
        jnp.dot(h, w_ref[...], preferred_element_type=jnp.float32)
        + b_ref[...]
    )


def _comb1(sums128, cnt8, w_bd, b128):
    return pl.pallas_call(
        _comb1_kernel,
        grid=(N_PAD // 8 // _CBLK,),
        in_specs=[
            pl.BlockSpec((N_CORES, _CBLK, 128), lambda i: (0, i, 0)),
            pl.BlockSpec((N_CORES, _CBLK, 8), lambda i: (0, i, 0)),
            pl.BlockSpec((128, 128), lambda i: (0, 0)),
            pl.BlockSpec((1, 128), lambda i: (0, 0)),
        ],
        out_specs=pl.BlockSpec((_CBLK, 128), lambda i: (i, 0)),
        out_shape=jax.ShapeDtypeStruct((N_PAD // 8, 128), jnp.float32),
    )(sums128, cnt8, w_bd, b128)


_FROWS = N_PAD // 16
_FOUT = N_NODES // 16


def _final_kernel(s_ref, c_ref, w_ref, b_ref, o_ref):
    s = s_ref[0] + s_ref[1]
    c = jnp.maximum(c_ref[0] + c_ref[1], 1.0)
    cb = jnp.repeat(c, HID, axis=1)
    h = s / cb
    y = (
        jnp.dot(h, w_ref[...], preferred_element_type=jnp.float32)
        + b_ref[...]
    )
    o_ref[...] = y[:_FOUT, :]


def _final(sums256, cnt16, w_bd, b32):
    return pl.pallas_call(
        _final_kernel,
        in_specs=[
            pl.BlockSpec((N_CORES, _FROWS, 256), lambda: (0, 0, 0)),
            pl.BlockSpec((N_CORES, _FROWS, HID), lambda: (0, 0, 0)),
            pl.BlockSpec((256, 32), lambda: (0, 0)),
            pl.BlockSpec((1, 32), lambda: (0, 0)),
        ],
        out_specs=pl.BlockSpec((_FOUT, 32), lambda: (0, 0)),
        out_shape=jax.ShapeDtypeStruct((_FOUT, 32), jnp.float32),
    )(sums256, cnt16, w_bd, b32)


def kernel(features, src_t2e, dst_t2e, src_e2t, dst_e2t, entity_embed,
           W_t2e_0, b_t2e_0, W_e2t_0, b_e2t_0,
           W_t2e_1, b_t2e_1, W_e2t_1, b_e2t_1,
           W_out, b_out):
    w1_bd = jnp.kron(jnp.eye(8, dtype=jnp.float32), W_e2t_1)
    b1_128 = jnp.tile(b_e2t_1, 8)[None, :]
    wout_bd = jnp.kron(jnp.eye(16, dtype=jnp.float32), W_out)
    bout_32 = jnp.tile(b_out, 16)[None, :]
    w0_bd = jnp.kron(jnp.eye(8, dtype=jnp.float32), W_t2e_0)
    b0_128 = jnp.tile(b_t2e_0, 8)[None, :]

    wh = _mm1(features.reshape(_MM1_ROWS, 1024), w0_bd,
              b0_128).reshape(N_NODES, HID)
    sums1, cnt1 = _seg_sum(wh, src_t2e, dst_t2e)
    wh1_128 = _comb1(sums1.reshape(N_CORES, N_PAD // 8, 128),
                     cnt1.reshape(N_CORES, N_PAD // 8, 8),
                     w1_bd, b1_128)
    wh1 = wh1_128.reshape(N_PAD, HID)
    sums2, cnt2 = _seg_sum(wh1, src_e2t, dst_e2t)
    out32 = _final(sums2.reshape(N_CORES, N_PAD // 16, 256),
                   cnt2.reshape(N_CORES, N_PAD // 16, HID),
                   wout_bd, bout_32)
    return out32.reshape(N_NODES, 2)


# speedup vs baseline: 56.8222x; 1.0762x over previous
import jax
import jax.numpy as jnp
from jax import lax
from jax.experimental import pallas as pl
from jax.experimental.pallas import tpu as pltpu
from jax.experimental.pallas import tpu_sc as plsc

N_NODES = 50000
N_PAD = 51200
N_EDGES = 1600000
HID = 16
N_CORES = 2
N_SUBCORES = 16
N_WORKERS = N_CORES * N_SUBCORES
E_PER_W = N_EDGES // N_WORKERS
CHUNK = 1000
N_CHUNKS = E_PER_W // CHUNK
UNROLL = 4
N_MAIN = 48
ROWS_PER_SUB = N_PAD // N_SUBCORES
IO_CHUNK = 400
N_IO = ROWS_PER_SUB // IO_CHUNK
ONES_BUF = 1024


def _make_seg_sum():
    mesh = plsc.VectorSubcoreMesh(
        core_axis_name="c", subcore_axis_name="s",
        num_cores=N_CORES, num_subcores=N_SUBCORES)

    def body(table, src, dst, sums_out, cnt_out,
             src_v, dst_v, rows_v, ones_v,
             acc_sh, cnt_sh,
             sem_l, sem_g, sem_s):
        cid = lax.axis_index("c")
        sid = lax.axis_index("s")
        wid = cid * N_SUBCORES + sid
        ebase = wid * E_PER_W

        zeros16 = jnp.zeros((16,), jnp.float32)
        ones16 = jnp.ones((16,), jnp.float32)

        def zrow(i, _):
            rows_v[0, i, :] = zeros16
            return 0
        lax.fori_loop(0, CHUNK, zrow, 0)

        def zrow1(i, _):
            ones_v[pl.ds(i * 16, 16)] = zeros16
            return 0
        lax.fori_loop(0, ONES_BUF // 16, zrow1, 0)

        rbase = sid * ROWS_PER_SUB
        for k in range(N_IO):
            pltpu.sync_copy(rows_v.at[0, pl.ds(0, IO_CHUNK)],
                            acc_sh.at[pl.ds(rbase + k * IO_CHUNK, IO_CHUNK)])
            pltpu.sync_copy(ones_v.at[pl.ds(0, IO_CHUNK)],
                            cnt_sh.at[pl.ds(rbase + k * IO_CHUNK, IO_CHUNK)])

        def orow(i, _):
            ones_v[pl.ds(i * 16, 16)] = ones16
            return 0
        lax.fori_loop(0, ONES_BUF // 16, orow, 0)
        plsc.subcore_barrier()

        def start_load(i, b):
            base = ebase + i * CHUNK
            pltpu.async_copy(src.at[pl.ds(base, CHUNK)], src_v.at[b],
                             sem_l.at[b])
            pltpu.async_copy(dst.at[pl.ds(base, CHUNK)], dst_v.at[b],
                             sem_l.at[b])

        def wait_load(i, b):
            base = ebase + i * CHUNK
            pltpu.make_async_copy(src.at[pl.ds(base, CHUNK)], src_v.at[b],
                                  sem_l.at[b]).wait()
            pltpu.make_async_copy(dst.at[pl.ds(base, CHUNK)], dst_v.at[b],
                                  sem_l.at[b]).wait()

        def start_scatter(b):
            pltpu.async_copy(rows_v.at[b], acc_sh.at[dst_v.at[b]],
                             sem_s.at[b], add=True)
            pltpu.async_copy(ones_v.at[pl.ds(0, CHUNK)],
                             cnt_sh.at[dst_v.at[b]],
                             sem_s.at[b], add=True)

        def wait_scatter(b):
            pltpu.make_async_copy(rows_v.at[b], acc_sh.at[dst_v.at[b]],
                                  sem_s.at[b]).wait()
            pltpu.make_async_copy(ones_v.at[pl.ds(0, CHUNK)],
                                  cnt_sh.at[dst_v.at[b]],
                                  sem_s.at[b]).wait()

        def start_gather(b):
            pltpu.async_copy(table.at[src_v.at[b]], rows_v.at[b],
                             sem_g.at[b])

        def wait_gather(b):
            pltpu.make_async_copy(table.at[src_v.at[b]], rows_v.at[b],
                                  sem_g.at[b]).wait()

        start_load(0, 0)
        start_load(1, 1)
        wait_load(0, 0)
        start_gather(0)

        def macro(m, _):
            for j in range(UNROLL):
                i = m * UNROLL + j

                @pl.when(i >= 2)
                def _():
                    wait_scatter((j + 2) % 4)

                start_load(i + 2, (j + 2) % 4)
                wait_load(i + 1, (j + 1) % 4)
                start_gather((j + 1) % 4)
                wait_gather(j)
                start_scatter(j)
            return 0
        lax.fori_loop(0, N_MAIN // UNROLL, macro, 0)

        wait_load(N_MAIN + 1, 1)
        start_gather(1)
        wait_gather(0)
        start_scatter(0)
        wait_gather(1)
        start_scatter(1)
        for b in (2, 3, 0, 1):
            wait_scatter(b)
        plsc.subcore_barrier()

        for k in range(N_IO):
            off = rbase + k * IO_CHUNK
            pltpu.sync_copy(acc_sh.at[pl.ds(off, IO_CHUNK)],
                            rows_v.at[0, pl.ds(0, IO_CHUNK)])
            pltpu.sync_copy(rows_v.at[0, pl.ds(0, IO_CHUNK)],
                            sums_out.at[cid, pl.ds(off, IO_CHUNK)])
            pltpu.sync_copy(cnt_sh.at[pl.ds(off, IO_CHUNK)],
                            ones_v.at[pl.ds(0, IO_CHUNK)])
            pltpu.sync_copy(ones_v.at[pl.ds(0, IO_CHUNK)],
                            cnt_out.at[cid, pl.ds(off, IO_CHUNK)])

    return pl.kernel(
        body,
        out_type=[
            jax.ShapeDtypeStruct((N_CORES, N_PAD, HID), jnp.float32),
            jax.ShapeDtypeStruct((N_CORES, N_PAD), jnp.float32),
        ],
        mesh=mesh,
        compiler_params=pltpu.CompilerParams(use_tc_tiling_on_sc=False),
        scratch_types=[
            pltpu.VMEM((UNROLL, CHUNK), jnp.int32),
            pltpu.VMEM((UNROLL, CHUNK), jnp.int32),
            pltpu.VMEM((UNROLL, CHUNK, HID), jnp.float32),
            pltpu.VMEM((ONES_BUF,), jnp.float32),
            pltpu.VMEM_SHARED((N_PAD, HID), jnp.float32),
            pltpu.VMEM_SHARED((N_PAD,), jnp.float32),
            pltpu.SemaphoreType.DMA((UNROLL,)),
            pltpu.SemaphoreType.DMA((UNROLL,)),
            pltpu.SemaphoreType.DMA((UNROLL,)),
        ],
    )


_seg_sum = _make_seg_sum()


_MM1_ROWS = N_NODES // 8
_MM1_OROWS = N_PAD // 8
_MM1_RBLK = 800


def _mm1_kernel(x_ref, w_ref, b_ref, o_ref):
    y = jnp.broadcast_to(b_ref[...], (_MM1_RBLK, 128))
    for a in range(8):
        y = y + jnp.dot(x_ref[:, a, :], w_ref[a],
                        preferred_element_type=jnp.float32)
    o_ref[...] = y


def _mm1(features8, w_bd, b128):
    return pl.pallas_call(
        _mm1_kernel,
        grid=(_MM1_OROWS // _MM1_RBLK,),
        in_specs=[
            pl.BlockSpec((_MM1_RBLK, 8, 128), lambda i: (i, 0, 0)),
            pl.BlockSpec((8, 128, 128), lambda i: (0, 0, 0)),
            pl.BlockSpec((1, 128), lambda i: (0, 0)),
        ],
        out_specs=pl.BlockSpec((_MM1_RBLK, 128), lambda i: (i, 0)),
        out_shape=jax.ShapeDtypeStruct((_MM1_OROWS, 128), jnp.float32),
    )(features8, w_bd, b128)


_CBLK = 3200


def _comb1_kernel(s_ref, c_ref, w_ref, b_ref, o_ref):
    s = s_ref[0] + s_ref[1]
    c = jnp.maximum(c_ref[0] + c_ref[1], 1.0)
    cb = jnp.repeat(c, HID, axis=1)
    h = s / cb
    h = jnp.where(h >= 0, h, 0.01 * h)
    o_ref[...] = (
        jnp.dot(h, w_ref[...], preferred_element_type=jnp.float32)
        + b_ref[...]
    )


def _comb1(sums128, cnt8, w_bd, b128):
    return pl.pallas_call(
        _comb1_kernel,
        grid=(N_PAD // 8 // _CBLK,),
        in_specs=[
            pl.BlockSpec((N_CORES, _CBLK, 128), lambda i: (0, i, 0)),
            pl.BlockSpec((N_CORES, _CBLK, 8), lambda i: (0, i, 0)),
            pl.BlockSpec((128, 128), lambda i: (0, 0)),
            pl.BlockSpec((1, 128), lambda i: (0, 0)),
        ],
        out_specs=pl.BlockSpec((_CBLK, 128), lambda i: (i, 0)),
        out_shape=jax.ShapeDtypeStruct((N_PAD // 8, 128), jnp.float32),
    )(sums128, cnt8, w_bd, b128)


_FROWS = N_PAD // 16
_FOUT = N_NODES // 16


def _final_kernel(s_ref, c_ref, w_ref, b_ref, o_ref):
    s = s_ref[0] + s_ref[1]
    c = jnp.maximum(c_ref[0] + c_ref[1], 1.0)
    cb = jnp.repeat(c, HID, axis=1)
    h = s / cb
    y = (
        jnp.dot(h, w_ref[...], preferred_element_type=jnp.float32)
        + b_ref[...]
    )
    o_ref[...] = y[:_FOUT, :]


def _final(sums256, cnt16, w_bd, b32):
    return pl.pallas_call(
        _final_kernel,
        in_specs=[
            pl.BlockSpec((N_CORES, _FROWS, 256), lambda: (0, 0, 0)),
            pl.BlockSpec((N_CORES, _FROWS, HID), lambda: (0, 0, 0)),
            pl.BlockSpec((256, 32), lambda: (0, 0)),
            pl.BlockSpec((1, 32), lambda: (0, 0)),
        ],
        out_specs=pl.BlockSpec((_FOUT, 32), lambda: (0, 0)),
        out_shape=jax.ShapeDtypeStruct((_FOUT, 32), jnp.float32),
    )(sums256, cnt16, w_bd, b32)


def kernel(features, src_t2e, dst_t2e, src_e2t, dst_e2t, entity_embed,
           W_t2e_0, b_t2e_0, W_e2t_0, b_e2t_0,
           W_t2e_1, b_t2e_1, W_e2t_1, b_e2t_1,
           W_out, b_out):
    w1_bd = jnp.kron(jnp.eye(8, dtype=jnp.float32), W_e2t_1)
    b1_128 = jnp.tile(b_e2t_1, 8)[None, :]
    wout_bd = jnp.kron(jnp.eye(16, dtype=jnp.float32), W_out)
    bout_32 = jnp.tile(b_out, 16)[None, :]
    w0_bd = jnp.kron(jnp.eye(8, dtype=jnp.float32), W_t2e_0)
    b0_128 = jnp.tile(b_t2e_0, 8)[None, :]

    wh = _mm1(features.reshape(_MM1_ROWS, 8, 128), w0_bd.reshape(8, 128, 128),
              b0_128).reshape(N_PAD, HID)
    sums1, cnt1 = _seg_sum(wh, src_t2e, dst_t2e)
    wh1_128 = _comb1(sums1.reshape(N_CORES, N_PAD // 8, 128),
                     cnt1.reshape(N_CORES, N_PAD // 8, 8),
                     w1_bd, b1_128)
    wh1 = wh1_128.reshape(N_PAD, HID)
    sums2, cnt2 = _seg_sum(wh1, src_e2t, dst_e2t)
    out32 = _final(sums2.reshape(N_CORES, N_PAD // 16, 256),
                   cnt2.reshape(N_CORES, N_PAD // 16, HID),
                   wout_bd, bout_32)
    return out32.reshape(N_NODES, 2)


# speedup vs baseline: 57.0814x; 1.0046x over previous
import jax
import jax.numpy as jnp
from jax import lax
from jax.experimental import pallas as pl
from jax.experimental.pallas import tpu as pltpu
from jax.experimental.pallas import tpu_sc as plsc

N_NODES = 50000
N_PAD = 51200
N_EDGES = 1600000
HID = 16
N_CORES = 2
N_SUBCORES = 16
N_WORKERS = N_CORES * N_SUBCORES
E_PER_W = N_EDGES // N_WORKERS
CHUNK = 1000
N_CHUNKS = E_PER_W // CHUNK
UNROLL = 4
N_MAIN = 48
ROWS_PER_SUB = N_PAD // N_SUBCORES
IO_CHUNK = 400
N_IO = ROWS_PER_SUB // IO_CHUNK
ONES_BUF = 1024


def _make_seg_sum():
    mesh = plsc.VectorSubcoreMesh(
        core_axis_name="c", subcore_axis_name="s",
        num_cores=N_CORES, num_subcores=N_SUBCORES)

    def body(table, src, dst, sums_out, cnt_out,
             src_v, dst_v, rows_v, ones_v,
             acc_sh, cnt_sh,
             sem_l, sem_g, sem_s):
        cid = lax.axis_index("c")
        sid = lax.axis_index("s")
        wid = cid * N_SUBCORES + sid
        ebase = wid * E_PER_W

        zeros16 = jnp.zeros((16,), jnp.float32)
        ones16 = jnp.ones((16,), jnp.float32)

        def zrow(i, _):
            rows_v[0, i, :] = zeros16
            return 0
        lax.fori_loop(0, CHUNK, zrow, 0)

        def zrow1(i, _):
            ones_v[pl.ds(i * 16, 16)] = zeros16
            return 0
        lax.fori_loop(0, ONES_BUF // 16, zrow1, 0)

        rbase = sid * ROWS_PER_SUB
        for k in range(N_IO):
            pltpu.sync_copy(rows_v.at[0, pl.ds(0, IO_CHUNK)],
                            acc_sh.at[pl.ds(rbase + k * IO_CHUNK, IO_CHUNK)])
            pltpu.sync_copy(ones_v.at[pl.ds(0, IO_CHUNK)],
                            cnt_sh.at[pl.ds(rbase + k * IO_CHUNK, IO_CHUNK)])

        def orow(i, _):
            ones_v[pl.ds(i * 16, 16)] = ones16
            return 0
        lax.fori_loop(0, ONES_BUF // 16, orow, 0)
        plsc.subcore_barrier()

        def start_load(i, b):
            base = ebase + i * CHUNK
            pltpu.async_copy(src.at[pl.ds(base, CHUNK)], src_v.at[b],
                             sem_l.at[b])
            pltpu.async_copy(dst.at[pl.ds(base, CHUNK)], dst_v.at[b],
                             sem_l.at[b])

        def wait_load(i, b):
            base = ebase + i * CHUNK
            pltpu.make_async_copy(src.at[pl.ds(base, CHUNK)], src_v.at[b],
                                  sem_l.at[b]).wait()
            pltpu.make_async_copy(dst.at[pl.ds(base, CHUNK)], dst_v.at[b],
                                  sem_l.at[b]).wait()

        def start_scatter(b):
            pltpu.async_copy(rows_v.at[b], acc_sh.at[dst_v.at[b]],
                             sem_s.at[b], add=True)
            pltpu.async_copy(ones_v.at[pl.ds(0, CHUNK)],
                             cnt_sh.at[dst_v.at[b]],
                             sem_s.at[b], add=True)

        def wait_scatter(b):
            pltpu.make_async_copy(rows_v.at[b], acc_sh.at[dst_v.at[b]],
                                  sem_s.at[b]).wait()
            pltpu.make_async_copy(ones_v.at[pl.ds(0, CHUNK)],
                                  cnt_sh.at[dst_v.at[b]],
                                  sem_s.at[b]).wait()

        def start_gather(b):
            pltpu.async_copy(table.at[src_v.at[b]], rows_v.at[b],
                             sem_g.at[b])

        def wait_gather(b):
            pltpu.make_async_copy(table.at[src_v.at[b]], rows_v.at[b],
                                  sem_g.at[b]).wait()

        start_load(0, 0)
        start_load(1, 1)
        wait_load(0, 0)
        start_gather(0)

        def macro(m, _):
            for j in range(UNROLL):
                i = m * UNROLL + j

                @pl.when(i >= 2)
                def _():
                    wait_scatter((j + 2) % 4)

                start_load(i + 2, (j + 2) % 4)
                wait_load(i + 1, (j + 1) % 4)
                start_gather((j + 1) % 4)
                wait_gather(j)
                start_scatter(j)
            return 0
        lax.fori_loop(0, N_MAIN // UNROLL, macro, 0)

        wait_load(N_MAIN + 1, 1)
        start_gather(1)
        wait_gather(0)
        start_scatter(0)
        wait_gather(1)
        start_scatter(1)
        for b in (2, 3, 0, 1):
            wait_scatter(b)
        plsc.subcore_barrier()

        for k in range(N_IO):
            off = rbase + k * IO_CHUNK
            pltpu.sync_copy(acc_sh.at[pl.ds(off, IO_CHUNK)],
                            rows_v.at[0, pl.ds(0, IO_CHUNK)])
            pltpu.sync_copy(rows_v.at[0, pl.ds(0, IO_CHUNK)],
                            sums_out.at[cid, pl.ds(off, IO_CHUNK)])
            pltpu.sync_copy(cnt_sh.at[pl.ds(off, IO_CHUNK)],
                            ones_v.at[pl.ds(0, IO_CHUNK)])
            pltpu.sync_copy(ones_v.at[pl.ds(0, IO_CHUNK)],
                            cnt_out.at[cid, pl.ds(off, IO_CHUNK)])

    return pl.kernel(
        body,
        out_type=[
            jax.ShapeDtypeStruct((N_CORES, N_PAD, HID), jnp.float32),
            jax.ShapeDtypeStruct((N_CORES, N_PAD), jnp.float32),
        ],
        mesh=mesh,
        compiler_params=pltpu.CompilerParams(use_tc_tiling_on_sc=False),
        scratch_types=[
            pltpu.VMEM((UNROLL, CHUNK), jnp.int32),
            pltpu.VMEM((UNROLL, CHUNK), jnp.int32),
            pltpu.VMEM((UNROLL, CHUNK, HID), jnp.float32),
            pltpu.VMEM((ONES_BUF,), jnp.float32),
            pltpu.VMEM_SHARED((N_PAD, HID), jnp.float32),
            pltpu.VMEM_SHARED((N_PAD,), jnp.float32),
            pltpu.SemaphoreType.DMA((UNROLL,)),
            pltpu.SemaphoreType.DMA((UNROLL,)),
            pltpu.SemaphoreType.DMA((UNROLL,)),
        ],
    )


_seg_sum = _make_seg_sum()


_MM1_ROWS = N_NODES // 8
_MM1_OROWS = N_PAD // 8
_MM1_RBLK = 800


def _mm1_kernel(x_ref, w_ref, b_ref, o_ref):
    y = jnp.broadcast_to(b_ref[...], (_MM1_RBLK, 128))
    for a in range(8):
        y = y + jnp.dot(x_ref[:, a, :], w_ref[a],
                        preferred_element_type=jnp.float32)
    o_ref[...] = y


def _mm1(features8, w_bd, b128):
    return pl.pallas_call(
        _mm1_kernel,
        grid=(_MM1_OROWS // _MM1_RBLK,),
        in_specs=[
            pl.BlockSpec((_MM1_RBLK, 8, 128), lambda i: (i, 0, 0)),
            pl.BlockSpec((8, 128, 128), lambda i: (0, 0, 0)),
            pl.BlockSpec((1, 128), lambda i: (0, 0)),
        ],
        out_specs=pl.BlockSpec((_MM1_RBLK, 128), lambda i: (i, 0)),
        out_shape=jax.ShapeDtypeStruct((_MM1_OROWS, 128), jnp.float32),
    )(features8, w_bd, b128)


_CBLK = 1600


def _comb1_kernel(s_ref, c_ref, w_ref, b_ref, o_ref):
    s = s_ref[0] + s_ref[1]
    c = jnp.maximum(c_ref[0] + c_ref[1], 1.0)
    cb = jnp.repeat(c, HID, axis=1)
    h = s / cb
    h = jnp.where(h >= 0, h, 0.01 * h)
    o_ref[...] = (
        jnp.dot(h, w_ref[...], preferred_element_type=jnp.float32)
        + b_ref[...]
    )


def _comb1(sums128, cnt8, w_bd, b128):
    return pl.pallas_call(
        _comb1_kernel,
        grid=(N_PAD // 8 // _CBLK,),
        in_specs=[
            pl.BlockSpec((N_CORES, _CBLK, 128), lambda i: (0, i, 0)),
            pl.BlockSpec((N_CORES, _CBLK, 8), lambda i: (0, i, 0)),
            pl.BlockSpec((128, 128), lambda i: (0, 0)),
            pl.BlockSpec((1, 128), lambda i: (0, 0)),
        ],
        out_specs=pl.BlockSpec((_CBLK, 128), lambda i: (i, 0)),
        out_shape=jax.ShapeDtypeStruct((N_PAD // 8, 128), jnp.float32),
    )(sums128, cnt8, w_bd, b128)


_FROWS = N_PAD // 16
_FOUT = N_NODES // 16


def _final_kernel(s_ref, c_ref, w_ref, b_ref, o_ref):
    s = s_ref[0] + s_ref[1]
    c = jnp.maximum(c_ref[0] + c_ref[1], 1.0)
    cb = jnp.repeat(c, HID, axis=1)
    h = s / cb
    y = (
        jnp.dot(h, w_ref[...], preferred_element_type=jnp.float32)
        + b_ref[...]
    )
    o_ref[...] = y[:_FOUT, :]


def _final(sums256, cnt16, w_bd, b32):
    return pl.pallas_call(
        _final_kernel,
        in_specs=[
            pl.BlockSpec((N_CORES, _FROWS, 256), lambda: (0, 0, 0)),
            pl.BlockSpec((N_CORES, _FROWS, HID), lambda: (0, 0, 0)),
            pl.BlockSpec((256, 32), lambda: (0, 0)),
            pl.BlockSpec((1, 32), lambda: (0, 0)),
        ],
        out_specs=pl.BlockSpec((_FOUT, 32), lambda: (0, 0)),
        out_shape=jax.ShapeDtypeStruct((_FOUT, 32), jnp.float32),
    )(sums256, cnt16, w_bd, b32)


def kernel(features, src_t2e, dst_t2e, src_e2t, dst_e2t, entity_embed,
           W_t2e_0, b_t2e_0, W_e2t_0, b_e2t_0,
           W_t2e_1, b_t2e_1, W_e2t_1, b_e2t_1,
           W_out, b_out):
    w1_bd = jnp.kron(jnp.eye(8, dtype=jnp.float32), W_e2t_1)
    b1_128 = jnp.tile(b_e2t_1, 8)[None, :]
    wout_bd = jnp.kron(jnp.eye(16, dtype=jnp.float32), W_out)
    bout_32 = jnp.tile(b_out, 16)[None, :]
    w0_bd = jnp.kron(jnp.eye(8, dtype=jnp.float32), W_t2e_0)
    b0_128 = jnp.tile(b_t2e_0, 8)[None, :]

    wh = _mm1(features.reshape(_MM1_ROWS, 8, 128), w0_bd.reshape(8, 128, 128),
              b0_128).reshape(N_PAD, HID)
    sums1, cnt1 = _seg_sum(wh, src_t2e, dst_t2e)
    wh1_128 = _comb1(sums1.reshape(N_CORES, N_PAD // 8, 128),
                     cnt1.reshape(N_CORES, N_PAD // 8, 8),
                     w1_bd, b1_128)
    wh1 = wh1_128.reshape(N_PAD, HID)
    sums2, cnt2 = _seg_sum(wh1, src_e2t, dst_e2t)
    out32 = _final(sums2.reshape(N_CORES, N_PAD // 16, 256),
                   cnt2.reshape(N_CORES, N_PAD // 16, HID),
                   wout_bd, bout_32)
    return out32.reshape(N_NODES, 2)


# speedup vs baseline: 57.5785x; 1.0087x over previous
import jax
import jax.numpy as jnp
from jax import lax
from jax.experimental import pallas as pl
from jax.experimental.pallas import tpu as pltpu
from jax.experimental.pallas import tpu_sc as plsc

N_NODES = 50000
N_PAD = 51200
N_EDGES = 1600000
HID = 16
N_CORES = 2
N_SUBCORES = 16
N_WORKERS = N_CORES * N_SUBCORES
E_PER_W = N_EDGES // N_WORKERS
CHUNK = 1000
N_CHUNKS = E_PER_W // CHUNK
UNROLL = 4
N_MAIN = 48
ROWS_PER_SUB = N_PAD // N_SUBCORES
IO_CHUNK = 400
N_IO = ROWS_PER_SUB // IO_CHUNK
ONES_BUF = 1024


def _make_seg_sum():
    mesh = plsc.VectorSubcoreMesh(
        core_axis_name="c", subcore_axis_name="s",
        num_cores=N_CORES, num_subcores=N_SUBCORES)

    def body(table, src, dst, sums_out, cnt_out,
             src_v, dst_v, rows_v, ones_v,
             acc_sh, cnt_sh,
             sem_l, sem_g, sem_s):
        cid = lax.axis_index("c")
        sid = lax.axis_index("s")
        wid = cid * N_SUBCORES + sid
        ebase = wid * E_PER_W

        zeros16 = jnp.zeros((16,), jnp.float32)
        ones16 = jnp.ones((16,), jnp.float32)

        def zrow(i, _):
            rows_v[0, i, :] = zeros16
            return 0
        lax.fori_loop(0, CHUNK, zrow, 0)

        def zrow1(i, _):
            ones_v[pl.ds(i * 16, 16)] = zeros16
            return 0
        lax.fori_loop(0, ONES_BUF // 16, zrow1, 0)

        rbase = sid * ROWS_PER_SUB
        for k in range(N_IO):
            pltpu.sync_copy(rows_v.at[0, pl.ds(0, IO_CHUNK)],
                            acc_sh.at[pl.ds(rbase + k * IO_CHUNK, IO_CHUNK)])
            pltpu.sync_copy(ones_v.at[pl.ds(0, IO_CHUNK)],
                            cnt_sh.at[pl.ds(rbase + k * IO_CHUNK, IO_CHUNK)])

        def orow(i, _):
            ones_v[pl.ds(i * 16, 16)] = ones16
            return 0
        lax.fori_loop(0, ONES_BUF // 16, orow, 0)
        plsc.subcore_barrier()

        def start_load(i, b):
            base = ebase + i * CHUNK
            pltpu.async_copy(src.at[pl.ds(base, CHUNK)], src_v.at[b],
                             sem_l.at[b])
            pltpu.async_copy(dst.at[pl.ds(base, CHUNK)], dst_v.at[b],
                             sem_l.at[b])

        def wait_load(i, b):
            base = ebase + i * CHUNK
            pltpu.make_async_copy(src.at[pl.ds(base, CHUNK)], src_v.at[b],
                                  sem_l.at[b]).wait()
            pltpu.make_async_copy(dst.at[pl.ds(base, CHUNK)], dst_v.at[b],
                                  sem_l.at[b]).wait()

        def start_scatter(b):
            pltpu.async_copy(rows_v.at[b], acc_sh.at[dst_v.at[b]],
                             sem_s.at[b], add=True)
            pltpu.async_copy(ones_v.at[pl.ds(0, CHUNK)],
                             cnt_sh.at[dst_v.at[b]],
                             sem_s.at[b], add=True)

        def wait_scatter(b):
            pltpu.make_async_copy(rows_v.at[b], acc_sh.at[dst_v.at[b]],
                                  sem_s.at[b]).wait()
            pltpu.make_async_copy(ones_v.at[pl.ds(0, CHUNK)],
                                  cnt_sh.at[dst_v.at[b]],
                                  sem_s.at[b]).wait()

        def start_gather(b):
            pltpu.async_copy(table.at[src_v.at[b]], rows_v.at[b],
                             sem_g.at[b])

        def wait_gather(b):
            pltpu.make_async_copy(table.at[src_v.at[b]], rows_v.at[b],
                                  sem_g.at[b]).wait()

        start_load(0, 0)
        start_load(1, 1)
        wait_load(0, 0)
        start_gather(0)

        def macro(m, _):
            for j in range(UNROLL):
                i = m * UNROLL + j

                @pl.when(i >= 2)
                def _():
                    wait_scatter((j + 2) % 4)

                start_load(i + 2, (j + 2) % 4)
                wait_load(i + 1, (j + 1) % 4)
                start_gather((j + 1) % 4)

                @pl.when(i >= 1)
                def _():
                    wait_gather((j + 3) % 4)
                    start_scatter((j + 3) % 4)
            return 0
        lax.fori_loop(0, N_MAIN // UNROLL, macro, 0)

        wait_load(N_MAIN + 1, 1)
        start_gather(1)
        wait_gather(3)
        start_scatter(3)
        wait_gather(0)
        start_scatter(0)
        wait_gather(1)
        start_scatter(1)
        for b in (2, 3, 0, 1):
            wait_scatter(b)
        plsc.subcore_barrier()

        for k in range(N_IO):
            off = rbase + k * IO_CHUNK
            pltpu.sync_copy(acc_sh.at[pl.ds(off, IO_CHUNK)],
                            rows_v.at[0, pl.ds(0, IO_CHUNK)])
            pltpu.sync_copy(rows_v.at[0, pl.ds(0, IO_CHUNK)],
                            sums_out.at[cid, pl.ds(off, IO_CHUNK)])
            pltpu.sync_copy(cnt_sh.at[pl.ds(off, IO_CHUNK)],
                            ones_v.at[pl.ds(0, IO_CHUNK)])
            pltpu.sync_copy(ones_v.at[pl.ds(0, IO_CHUNK)],
                            cnt_out.at[cid, pl.ds(off, IO_CHUNK)])

    return pl.kernel(
        body,
        out_type=[
            jax.ShapeDtypeStruct((N_CORES, N_PAD, HID), jnp.float32),
            jax.ShapeDtypeStruct((N_CORES, N_PAD), jnp.float32),
        ],
        mesh=mesh,
        compiler_params=pltpu.CompilerParams(use_tc_tiling_on_sc=False),
        scratch_types=[
            pltpu.VMEM((UNROLL, CHUNK), jnp.int32),
            pltpu.VMEM((UNROLL, CHUNK), jnp.int32),
            pltpu.VMEM((UNROLL, CHUNK, HID), jnp.float32),
            pltpu.VMEM((ONES_BUF,), jnp.float32),
            pltpu.VMEM_SHARED((N_PAD, HID), jnp.float32),
            pltpu.VMEM_SHARED((N_PAD,), jnp.float32),
            pltpu.SemaphoreType.DMA((UNROLL,)),
            pltpu.SemaphoreType.DMA((UNROLL,)),
            pltpu.SemaphoreType.DMA((UNROLL,)),
        ],
    )


_seg_sum = _make_seg_sum()


_MM1_ROWS = N_NODES // 8
_MM1_OROWS = N_PAD // 8
_MM1_RBLK = 800


def _mm1_kernel(x_ref, w_ref, b_ref, o_ref):
    y = jnp.broadcast_to(b_ref[...], (_MM1_RBLK, 128))
    for a in range(8):
        y = y + jnp.dot(x_ref[:, a, :], w_ref[a],
                        preferred_element_type=jnp.float32)
    o_ref[...] = y


def _mm1(features8, w_bd, b128):
    return pl.pallas_call(
        _mm1_kernel,
        grid=(_MM1_OROWS // _MM1_RBLK,),
        in_specs=[
            pl.BlockSpec((_MM1_RBLK, 8, 128), lambda i: (i, 0, 0)),
            pl.BlockSpec((8, 128, 128), lambda i: (0, 0, 0)),
            pl.BlockSpec((1, 128), lambda i: (0, 0)),
        ],
        out_specs=pl.BlockSpec((_MM1_RBLK, 128), lambda i: (i, 0)),
        out_shape=jax.ShapeDtypeStruct((_MM1_OROWS, 128), jnp.float32),
    )(features8, w_bd, b128)


_CBLK = 1600


def _comb1_kernel(s_ref, c_ref, w_ref, b_ref, o_ref):
    s = s_ref[0] + s_ref[1]
    c = jnp.maximum(c_ref[0] + c_ref[1], 1.0)
    cb = jnp.repeat(c, HID, axis=1)
    h = s / cb
    h = jnp.where(h >= 0, h, 0.01 * h)
    o_ref[...] = (
        jnp.dot(h, w_ref[...], preferred_element_type=jnp.float32)
        + b_ref[...]
    )


def _comb1(sums128, cnt8, w_bd, b128):
    return pl.pallas_call(
        _comb1_kernel,
        grid=(N_PAD // 8 // _CBLK,),
        in_specs=[
            pl.BlockSpec((N_CORES, _CBLK, 128), lambda i: (0, i, 0)),
            pl.BlockSpec((N_CORES, _CBLK, 8), lambda i: (0, i, 0)),
            pl.BlockSpec((128, 128), lambda i: (0, 0)),
            pl.BlockSpec((1, 128), lambda i: (0, 0)),
        ],
        out_specs=pl.BlockSpec((_CBLK, 128), lambda i: (i, 0)),
        out_shape=jax.ShapeDtypeStruct((N_PAD // 8, 128), jnp.float32),
    )(sums128, cnt8, w_bd, b128)


_FROWS = N_PAD // 16
_FOUT = N_NODES // 16


def _final_kernel(s_ref, c_ref, w_ref, b_ref, o_ref):
    s = s_ref[0] + s_ref[1]
    c = jnp.maximum(c_ref[0] + c_ref[1], 1.0)
    cb = jnp.repeat(c, HID, axis=1)
    h = s / cb
    y = (
        jnp.dot(h, w_ref[...], preferred_element_type=jnp.float32)
        + b_ref[...]
    )
    o_ref[...] = y[:_FOUT, :]


def _final(sums256, cnt16, w_bd, b32):
    return pl.pallas_call(
        _final_kernel,
        in_specs=[
            pl.BlockSpec((N_CORES, _FROWS, 256), lambda: (0, 0, 0)),
            pl.BlockSpec((N_CORES, _FROWS, HID), lambda: (0, 0, 0)),
            pl.BlockSpec((256, 32), lambda: (0, 0)),
            pl.BlockSpec((1, 32), lambda: (0, 0)),
        ],
        out_specs=pl.BlockSpec((_FOUT, 32), lambda: (0, 0)),
        out_shape=jax.ShapeDtypeStruct((_FOUT, 32), jnp.float32),
    )(sums256, cnt16, w_bd, b32)


def kernel(features, src_t2e, dst_t2e, src_e2t, dst_e2t, entity_embed,
           W_t2e_0, b_t2e_0, W_e2t_0, b_e2t_0,
           W_t2e_1, b_t2e_1, W_e2t_1, b_e2t_1,
           W_out, b_out):
    w1_bd = jnp.kron(jnp.eye(8, dtype=jnp.float32), W_e2t_1)
    b1_128 = jnp.tile(b_e2t_1, 8)[None, :]
    wout_bd = jnp.kron(jnp.eye(16, dtype=jnp.float32), W_out)
    bout_32 = jnp.tile(b_out, 16)[None, :]
    w0_bd = jnp.kron(jnp.eye(8, dtype=jnp.float32), W_t2e_0)
    b0_128 = jnp.tile(b_t2e_0, 8)[None, :]

    wh = _mm1(features.reshape(_MM1_ROWS, 8, 128), w0_bd.reshape(8, 128, 128),
              b0_128).reshape(N_PAD, HID)
    sums1, cnt1 = _seg_sum(wh, src_t2e, dst_t2e)
    wh1_128 = _comb1(sums1.reshape(N_CORES, N_PAD // 8, 128),
                     cnt1.reshape(N_CORES, N_PAD // 8, 8),
                     w1_bd, b1_128)
    wh1 = wh1_128.reshape(N_PAD, HID)
    sums2, cnt2 = _seg_sum(wh1, src_e2t, dst_e2t)
    out32 = _final(sums2.reshape(N_CORES, N_PAD // 16, 256),
                   cnt2.reshape(N_CORES, N_PAD // 16, HID),
                   wout_bd, bout_32)
    return out32.reshape(N_NODES, 2)
